# baseline (speedup 1.0000x reference)
import functools

import jax
import jax.numpy as jnp
from jax import lax
from jax.experimental import pallas as pl
from jax.experimental.pallas import tpu as pltpu
from jax.experimental.pallas import tpu_sc as plsc

NC = 2
NS = 16
NW = NC * NS

CHUNK = 128


def _gather_body(n_chunks, uids_hbm, keys_hbm, out_hbm, idx_v, rows_v, gsem):
    wid = lax.axis_index("c") * NS + lax.axis_index("s")
    base = wid * (n_chunks * CHUNK)
    pltpu.sync_copy(uids_hbm.at[wid], idx_v)

    def step(c, carry):
        pltpu.async_copy(keys_hbm.at[idx_v.at[c]], rows_v, gsem).wait()
        pltpu.sync_copy(rows_v, out_hbm.at[pl.ds(base + c * CHUNK, CHUNK)])
        return carry

    lax.fori_loop(0, n_chunks, step, 0, unroll=False)


def kernel(uids, keys):
    batch, hist = uids.shape
    n_rows = batch * hist
    key_dim = keys.shape[1]
    assert n_rows % (NW * CHUNK) == 0
    n_chunks = n_rows // (NW * CHUNK)

    uids_w = uids.reshape(NW, n_chunks, CHUNK)
    mesh = plsc.VectorSubcoreMesh(core_axis_name="c", subcore_axis_name="s")
    flat = pl.kernel(
        functools.partial(_gather_body, n_chunks),
        out_type=jax.ShapeDtypeStruct((n_rows, key_dim), keys.dtype),
        mesh=mesh,
        scratch_types=[
            pltpu.VMEM((n_chunks, CHUNK), jnp.int32),
            pltpu.VMEM((CHUNK, key_dim), keys.dtype),
            pltpu.SemaphoreType.DMA,
        ],
        compiler_params=pltpu.CompilerParams(use_tc_tiling_on_sc=False),
    )(uids_w, keys)
    return flat.reshape(batch, hist, key_dim)


# speedup vs baseline: 1.0414x; 1.0414x over previous
import functools

import jax
import jax.numpy as jnp
from jax import lax
from jax.experimental import pallas as pl
from jax.experimental.pallas import tpu as pltpu
from jax.experimental.pallas import tpu_sc as plsc

NC = 2
NS = 16
NW = NC * NS

CHUNK = 128
G = 5


def _gather_body(n_chunks, uids_hbm, keys_hbm, out_hbm, idx_v, rows_v, gsem,
                 osem):
    wid = lax.axis_index("c") * NS + lax.axis_index("s")
    rows_per_w = n_chunks * CHUNK
    base = wid * rows_per_w
    ng = n_chunks // G
    grp = G * CHUNK
    pltpu.sync_copy(uids_hbm.at[wid], idx_v)

    def fire(g):
        b = lax.rem(g, 2)
        for j in range(G):
            pltpu.async_copy(
                keys_hbm.at[idx_v.at[g * G + j]],
                rows_v.at[b, pl.ds(j * CHUNK, CHUNK)],
                gsem,
            )

    def wait_gathers():
        pltpu.make_async_copy(
            keys_hbm.at[pl.ds(0, grp)], rows_v.at[0], gsem
        ).wait()

    def start_out(g):
        b = lax.rem(g, 2)
        pltpu.async_copy(
            rows_v.at[b], out_hbm.at[pl.ds(base + g * grp, grp)], osem
        )

    def wait_out():
        pltpu.make_async_copy(
            rows_v.at[0], out_hbm.at[pl.ds(base, grp)], osem
        ).wait()

    fire(0)

    def step(g, carry):
        wait_gathers()
        start_out(g)

        @pl.when((g + 1 < ng) & (g >= 1))
        def _():
            wait_out()

        @pl.when(g + 1 < ng)
        def _():
            fire(g + 1)

        return carry

    lax.fori_loop(0, ng, step, 0, unroll=False)
    wait_out()
    if ng >= 2:
        wait_out()


def kernel(uids, keys):
    batch, hist = uids.shape
    n_rows = batch * hist
    key_dim = keys.shape[1]
    assert n_rows % (NW * CHUNK * G) == 0
    n_chunks = n_rows // (NW * CHUNK)

    uids_w = uids.reshape(NW, n_chunks, CHUNK)
    mesh = plsc.VectorSubcoreMesh(core_axis_name="c", subcore_axis_name="s")
    flat = pl.kernel(
        functools.partial(_gather_body, n_chunks),
        out_type=jax.ShapeDtypeStruct((n_rows, key_dim), keys.dtype),
        mesh=mesh,
        scratch_types=[
            pltpu.VMEM((n_chunks, CHUNK), jnp.int32),
            pltpu.VMEM((2, G * CHUNK, key_dim), keys.dtype),
            pltpu.SemaphoreType.DMA,
            pltpu.SemaphoreType.DMA,
        ],
        compiler_params=pltpu.CompilerParams(use_tc_tiling_on_sc=False),
    )(uids_w, keys)
    return flat.reshape(batch, hist, key_dim)


# speedup vs baseline: 1.0436x; 1.0021x over previous
import functools

import jax
import jax.numpy as jnp
from jax import lax
from jax.experimental import pallas as pl
from jax.experimental.pallas import tpu as pltpu
from jax.experimental.pallas import tpu_sc as plsc

NC = 2
NS = 16
NW = NC * NS

NB = 8
AHEAD = 6


def _gather_body(rows_per_w, uids_hbm, keys_hbm, out_hbm, idx_v, rows_v,
                 gsem, osem):
    wid = lax.axis_index("c") * NS + lax.axis_index("s")
    base = wid * rows_per_w
    hist = idx_v.shape[1]
    key_dim = rows_v.shape[2]
    pltpu.sync_copy(uids_hbm.at[pl.ds(base, rows_per_w)], idx_v)

    def fire(r):
        pltpu.async_copy(
            keys_hbm.at[idx_v.at[r]], rows_v.at[lax.rem(r, NB)], gsem
        )

    def wait_gather():
        pltpu.make_async_copy(
            keys_hbm.at[pl.ds(0, hist)], rows_v.at[0], gsem
        ).wait()

    def start_out(r):
        pltpu.async_copy(
            rows_v.at[lax.rem(r, NB)], out_hbm.at[base + r], osem
        )

    def wait_out():
        pltpu.make_async_copy(
            rows_v.at[0], out_hbm.at[0], osem
        ).wait()

    for r in range(AHEAD):
        fire(r)

    def step(r, carry):
        wait_gather()
        start_out(r)

        @pl.when(r >= 2)
        def _():
            wait_out()

        @pl.when(r + AHEAD < rows_per_w)
        def _():
            fire(r + AHEAD)

        return carry

    lax.fori_loop(0, rows_per_w, step, 0, unroll=False)
    wait_out()
    wait_out()


def kernel(uids, keys):
    batch, hist = uids.shape
    key_dim = keys.shape[1]
    assert batch % NW == 0
    rows_per_w = batch // NW

    mesh = plsc.VectorSubcoreMesh(core_axis_name="c", subcore_axis_name="s")
    return pl.kernel(
        functools.partial(_gather_body, rows_per_w),
        out_type=jax.ShapeDtypeStruct((batch, hist, key_dim), keys.dtype),
        mesh=mesh,
        scratch_types=[
            pltpu.VMEM((rows_per_w, hist), jnp.int32),
            pltpu.VMEM((NB, hist, key_dim), keys.dtype),
            pltpu.SemaphoreType.DMA,
            pltpu.SemaphoreType.DMA,
        ],
        compiler_params=pltpu.CompilerParams(use_tc_tiling_on_sc=False),
    )(uids, keys)
